# baseline (speedup 1.0000x reference)
import functools
import math

import jax
import jax.numpy as jnp
from jax import lax
from jax.experimental import pallas as pl
from jax.experimental.pallas import tpu as pltpu
from jax.experimental.pallas import tpu_sc as plsc

_D = 32
_L = 16
_NC, _NS = 2, 16
_NW = _NC * _NS
_V = 1000000
_SCALE = math.sqrt(32.0)

_PANEL = 128
_NFULL = _V // _PANEL
_TAIL = _V - _NFULL * _PANEL
_A_PER_W = -(-_NFULL // _NW)
_A_NBUF = 2

_CHUNK = 1280
_B_NBUF = 2


def _transpose_scale_table(table_t):
    mesh = plsc.VectorSubcoreMesh(
        core_axis_name="c", subcore_axis_name="s",
        num_cores=_NC, num_subcores=_NS)

    @functools.partial(
        pl.kernel,
        out_type=jax.ShapeDtypeStruct((_V * _D,), jnp.float32),
        mesh=mesh,
        scratch_types=[
            pltpu.VMEM((_A_NBUF, _D, _PANEL), jnp.float32),
            pltpu.VMEM((_A_NBUF, _D * _PANEL), jnp.float32),
            pltpu.VMEM((_D, _TAIL), jnp.float32),
            pltpu.VMEM((_D * _TAIL,), jnp.float32),
            pltpu.SemaphoreType.DMA,
            pltpu.SemaphoreType.DMA,
            pltpu.SemaphoreType.DMA,
            pltpu.SemaphoreType.DMA,
        ],
        compiler_params=pltpu.CompilerParams(needs_layout_passes=False),
    )
    def conv(tin, tout, panel, stage, tpanel, tstage, sg0, sg1, so0, so1):
        sg = (sg0, sg1)
        so = (so0, so1)
        wid = lax.axis_index("s") * _NC + lax.axis_index("c")
        c_lo = wid * _A_PER_W
        c_hi = jnp.minimum(c_lo + _A_PER_W, _NFULL)
        n_c = c_hi - c_lo
        iota = lax.iota(jnp.int32, _L)

        def load_desc(c, b):
            return pltpu.make_async_copy(
                tin.at[:, pl.ds(c * _PANEL, _PANEL)], panel.at[b], sg[b])

        def store_desc(c, b):
            return pltpu.make_async_copy(
                stage.at[b], tout.at[pl.ds(c * (_D * _PANEL), _D * _PANEL)],
                so[b])

        def transpose_panel(b):
            def row(r, carry):
                base = r * _D
                for h in range(_D // _L):
                    v = plsc.load_gather(
                        panel.at[b], [iota + (h * _L), jnp.full((_L,), r, jnp.int32)])
                    stage[b, pl.ds(base + h * _L, _L)] = v * _SCALE
                return carry
            lax.fori_loop(0, _PANEL, row, 0)

        @pl.when(n_c > 0)
        def _():
            for b in range(_A_NBUF):
                @pl.when(b < n_c)
                def _():
                    load_desc(c_lo + b, b).start()

            def step(i, carry):
                b = lax.rem(i, _A_NBUF)

                def go(b, i=i):
                    c = c_lo + i
                    load_desc(c, b).wait()
                    transpose_panel(b)
                    sd = store_desc(c, b)
                    sd.start()
                    sd.wait()

                    @pl.when(i + _A_NBUF < n_c)
                    def _():
                        load_desc(c + _A_NBUF, b).start()

                for bb in range(_A_NBUF):
                    @pl.when(b == bb)
                    def _(bb=bb):
                        go(bb)
                return carry

            lax.fori_loop(0, n_c, step, 0)

        @pl.when(wid == _NW - 1)
        def _():
            pltpu.sync_copy(tin.at[:, pl.ds(_NFULL * _PANEL, _TAIL)], tpanel)

            def row(r, carry):
                base = r * _D
                for h in range(_D // _L):
                    v = plsc.load_gather(
                        tpanel, [iota + (h * _L), jnp.full((_L,), r, jnp.int32)])
                    tstage[pl.ds(base + h * _L, _L)] = v * _SCALE
                return carry
            lax.fori_loop(0, _TAIL, row, 0)
            pltpu.sync_copy(
                tstage, tout.at[pl.ds(_NFULL * _PANEL * _D, _TAIL * _D)])

    return conv(table_t)


def _gather_rows(table_rm, idx_flat, n):
    per_w = n // _NW
    nchunk = per_w // _CHUNK
    nsuper = nchunk // _B_NBUF
    assert per_w * _NW == n and nchunk * _CHUNK == per_w
    assert nsuper * _B_NBUF == nchunk and nsuper >= 2

    mesh = plsc.VectorSubcoreMesh(
        core_axis_name="c", subcore_axis_name="s",
        num_cores=_NC, num_subcores=_NS)

    @functools.partial(
        pl.kernel,
        out_type=jax.ShapeDtypeStruct((n, _D), jnp.float32),
        mesh=mesh,
        scratch_types=[
            pltpu.VMEM((per_w,), jnp.int32),
            pltpu.VMEM((_B_NBUF, _CHUNK, _D), jnp.float32),
            pltpu.SemaphoreType.DMA,
            pltpu.SemaphoreType.DMA,
            pltpu.SemaphoreType.DMA,
            pltpu.SemaphoreType.DMA,
        ],
        compiler_params=pltpu.CompilerParams(use_tc_tiling_on_sc=False),
    )
    def emb(idx_hbm, table_hbm, out_hbm, idx_all, rows, sg0, sg1, so0, so1):
        sg = (sg0, sg1)
        so = (so0, so1)
        wid = lax.axis_index("s") * _NC + lax.axis_index("c")
        w_base = wid * per_w
        pltpu.sync_copy(idx_hbm.at[pl.ds(w_base, per_w)], idx_all)

        def gather_desc(c, b):
            return pltpu.make_async_copy(
                table_hbm.at[idx_all.at[pl.ds(c * _CHUNK, _CHUNK)]],
                rows.at[b], sg[b])

        def store_desc(c, b):
            return pltpu.make_async_copy(
                rows.at[b], out_hbm.at[pl.ds(w_base + c * _CHUNK, _CHUNK)],
                so[b])

        for b in range(_B_NBUF):
            gather_desc(b, b).start()

        @pl.loop(0, nsuper - 1)
        def super_step(g):
            for b in range(_B_NBUF):
                c = g * _B_NBUF + b
                gather_desc(c, b).wait()
                sd = store_desc(c, b)
                sd.start()
                sd.wait()
                gather_desc(c + _B_NBUF, b).start()

        for b in range(_B_NBUF):
            c_last = (nsuper - 1) * _B_NBUF + b
            gather_desc(c_last, b).wait()
            store_desc(c_last, b).start()
        for b in range(_B_NBUF):
            store_desc(nchunk - _B_NBUF + b, b).wait()

    return emb(idx_flat, table_rm)


def kernel(encoded_data, embedding_table):
    batch, seqlen = encoded_data.shape
    n = batch * seqlen

    table_t = jnp.transpose(embedding_table)
    table_flat = _transpose_scale_table(table_t)
    table_rm = jnp.reshape(table_flat, (_V, _D))

    idx_flat = encoded_data.reshape(n).astype(jnp.int32)
    out = _gather_rows(table_rm, idx_flat, n)
    return out.reshape(batch, seqlen, _D)


# speedup vs baseline: 1.0999x; 1.0999x over previous
import functools
import math

import jax
import jax.numpy as jnp
from jax import lax
from jax.experimental import pallas as pl
from jax.experimental.pallas import tpu as pltpu
from jax.experimental.pallas import tpu_sc as plsc

_D = 32
_L = 16
_NC, _NS = 2, 16
_NW = _NC * _NS
_V = 1000000
_SCALE = math.sqrt(32.0)

_PANEL = 128
_NFULL = _V // _PANEL
_TAIL = _V - _NFULL * _PANEL
_A_PER_W = -(-_NFULL // _NW)
_A_NBUF = 2

_CHUNK = 1280
_B_NBUF = 2


def _transpose_scale_table(table_t):
    mesh = plsc.VectorSubcoreMesh(
        core_axis_name="c", subcore_axis_name="s",
        num_cores=_NC, num_subcores=_NS)

    @functools.partial(
        pl.kernel,
        out_type=jax.ShapeDtypeStruct((_V * _D,), jnp.float32),
        mesh=mesh,
        scratch_types=[
            pltpu.VMEM((_D, _PANEL), jnp.float32),
            pltpu.VMEM((_D, _PANEL), jnp.float32),
            pltpu.VMEM((_D * _PANEL,), jnp.float32),
            pltpu.VMEM((_D * _PANEL,), jnp.float32),
            pltpu.VMEM((_D, _TAIL), jnp.float32),
            pltpu.VMEM((_D * _TAIL,), jnp.float32),
            pltpu.SemaphoreType.DMA,
            pltpu.SemaphoreType.DMA,
            pltpu.SemaphoreType.DMA,
            pltpu.SemaphoreType.DMA,
        ],
        compiler_params=pltpu.CompilerParams(needs_layout_passes=False),
    )
    def conv(tin, tout, panel0, panel1, stage0, stage1, tpanel, tstage,
             sg0, sg1, so0, so1):
        panel = (panel0, panel1)
        stage = (stage0, stage1)
        sg = (sg0, sg1)
        so = (so0, so1)
        wid = lax.axis_index("s") * _NC + lax.axis_index("c")
        c_lo = wid * _A_PER_W
        c_hi = jnp.minimum(c_lo + _A_PER_W, _NFULL)
        n_c = c_hi - c_lo
        iota = lax.iota(jnp.int32, _L)
        iota_d = iota * _D

        def load_desc(c, b):
            return pltpu.make_async_copy(
                tin.at[:, pl.ds(c * _PANEL, _PANEL)], panel[b], sg[b])

        def store_desc(c, b):
            return pltpu.make_async_copy(
                stage[b], tout.at[pl.ds(c * (_D * _PANEL), _D * _PANEL)],
                so[b])

        def transpose_panel(b):
            for d in range(_D):
                for r0 in range(_PANEL // _L):
                    v = panel[b][d, pl.ds(r0 * _L, _L)]
                    plsc.store_scatter(
                        stage[b], [iota_d + (r0 * _L * _D + d)], v * _SCALE)

        @pl.when(n_c > 0)
        def _():
            for b in range(_A_NBUF):
                @pl.when(b < n_c)
                def _():
                    load_desc(c_lo + b, b).start()

            def step(i, carry):
                b = lax.rem(i, _A_NBUF)

                def go(b, i=i):
                    c = c_lo + i
                    load_desc(c, b).wait()
                    transpose_panel(b)
                    sd = store_desc(c, b)
                    sd.start()
                    sd.wait()

                    @pl.when(i + _A_NBUF < n_c)
                    def _():
                        load_desc(c + _A_NBUF, b).start()

                for bb in range(_A_NBUF):
                    @pl.when(b == bb)
                    def _(bb=bb):
                        go(bb)
                return carry

            lax.fori_loop(0, n_c, step, 0)

        @pl.when(wid == _NW - 1)
        def _():
            pltpu.sync_copy(tin.at[:, pl.ds(_NFULL * _PANEL, _TAIL)], tpanel)
            for d in range(_D):
                for r0 in range(_TAIL // _L):
                    v = tpanel[d, pl.ds(r0 * _L, _L)]
                    plsc.store_scatter(
                        tstage, [iota_d + (r0 * _L * _D + d)], v * _SCALE)
            pltpu.sync_copy(
                tstage, tout.at[pl.ds(_NFULL * _PANEL * _D, _TAIL * _D)])

    return conv(table_t)


def _gather_rows(table_rm, idx_flat, n):
    per_w = n // _NW
    nchunk = per_w // _CHUNK
    nsuper = nchunk // _B_NBUF
    assert per_w * _NW == n and nchunk * _CHUNK == per_w
    assert nsuper * _B_NBUF == nchunk and nsuper >= 2

    mesh = plsc.VectorSubcoreMesh(
        core_axis_name="c", subcore_axis_name="s",
        num_cores=_NC, num_subcores=_NS)

    @functools.partial(
        pl.kernel,
        out_type=jax.ShapeDtypeStruct((n, _D), jnp.float32),
        mesh=mesh,
        scratch_types=[
            pltpu.VMEM((per_w,), jnp.int32),
            pltpu.VMEM((_B_NBUF, _CHUNK, _D), jnp.float32),
            pltpu.SemaphoreType.DMA,
            pltpu.SemaphoreType.DMA,
            pltpu.SemaphoreType.DMA,
            pltpu.SemaphoreType.DMA,
        ],
        compiler_params=pltpu.CompilerParams(use_tc_tiling_on_sc=False),
    )
    def emb(idx_hbm, table_hbm, out_hbm, idx_all, rows, sg0, sg1, so0, so1):
        sg = (sg0, sg1)
        so = (so0, so1)
        wid = lax.axis_index("s") * _NC + lax.axis_index("c")
        w_base = wid * per_w
        pltpu.sync_copy(idx_hbm.at[pl.ds(w_base, per_w)], idx_all)

        def gather_desc(c, b):
            return pltpu.make_async_copy(
                table_hbm.at[idx_all.at[pl.ds(c * _CHUNK, _CHUNK)]],
                rows.at[b], sg[b])

        def store_desc(c, b):
            return pltpu.make_async_copy(
                rows.at[b], out_hbm.at[pl.ds(w_base + c * _CHUNK, _CHUNK)],
                so[b])

        for b in range(_B_NBUF):
            gather_desc(b, b).start()

        @pl.loop(0, nsuper - 1)
        def super_step(g):
            for b in range(_B_NBUF):
                c = g * _B_NBUF + b
                gather_desc(c, b).wait()
                sd = store_desc(c, b)
                sd.start()
                sd.wait()
                gather_desc(c + _B_NBUF, b).start()

        for b in range(_B_NBUF):
            c_last = (nsuper - 1) * _B_NBUF + b
            gather_desc(c_last, b).wait()
            store_desc(c_last, b).start()
        for b in range(_B_NBUF):
            store_desc(nchunk - _B_NBUF + b, b).wait()

    return emb(idx_flat, table_rm)


def kernel(encoded_data, embedding_table):
    batch, seqlen = encoded_data.shape
    n = batch * seqlen

    table_t = jnp.transpose(embedding_table)
    table_flat = _transpose_scale_table(table_t)
    table_rm = jnp.reshape(table_flat, (_V, _D))

    idx_flat = encoded_data.reshape(n).astype(jnp.int32)
    out = _gather_rows(table_rm, idx_flat, n)
    return out.reshape(batch, seqlen, _D)


# speedup vs baseline: 2.1005x; 1.9097x over previous
import functools
import math

import jax
import jax.numpy as jnp
from jax import lax
from jax.experimental import pallas as pl
from jax.experimental.pallas import tpu as pltpu
from jax.experimental.pallas import tpu_sc as plsc

_D = 32
_L = 16
_NC, _NS = 2, 16
_NW = _NC * _NS
_V = 1000000
_SCALE = math.sqrt(32.0)

_PANEL = 128
_NFULL = _V // _PANEL
_TAIL = _V - _NFULL * _PANEL
_A_PER_W = -(-_NFULL // _NW)
_A_NBUF = 2

_CHUNK = 1280
_B_NBUF = 2


def _transpose_scale_table(table_t):
    mesh = plsc.VectorSubcoreMesh(
        core_axis_name="c", subcore_axis_name="s",
        num_cores=_NC, num_subcores=_NS)

    @functools.partial(
        pl.kernel,
        out_type=jax.ShapeDtypeStruct((_V * _D,), jnp.float32),
        mesh=mesh,
        scratch_types=[
            pltpu.VMEM((_D, _PANEL), jnp.float32),
            pltpu.VMEM((_D, _PANEL), jnp.float32),
            pltpu.VMEM((_D * _PANEL,), jnp.float32),
            pltpu.VMEM((_D * _PANEL,), jnp.float32),
            pltpu.VMEM((_D, _TAIL), jnp.float32),
            pltpu.VMEM((_D * _TAIL,), jnp.float32),
            pltpu.SemaphoreType.DMA,
            pltpu.SemaphoreType.DMA,
            pltpu.SemaphoreType.DMA,
            pltpu.SemaphoreType.DMA,
        ],
        compiler_params=pltpu.CompilerParams(needs_layout_passes=False),
    )
    def conv(tin, tout, panel0, panel1, stage0, stage1, tpanel, tstage,
             sg0, sg1, so0, so1):
        panel = (panel0, panel1)
        stage = (stage0, stage1)
        sg = (sg0, sg1)
        so = (so0, so1)
        wid = lax.axis_index("s") * _NC + lax.axis_index("c")
        c_lo = wid * _A_PER_W
        c_hi = jnp.minimum(c_lo + _A_PER_W, _NFULL)
        n_c = c_hi - c_lo
        iota = lax.iota(jnp.int32, _L)
        iota_d = iota * _D

        def load_desc(c, b):
            return pltpu.make_async_copy(
                tin.at[:, pl.ds(c * _PANEL, _PANEL)], panel[b], sg[b])

        def store_desc(c, b):
            return pltpu.make_async_copy(
                stage[b], tout.at[pl.ds(c * (_D * _PANEL), _D * _PANEL)],
                so[b])

        def transpose_panel(b):
            @functools.partial(plsc.parallel_loop, 0, _PANEL, unroll=8)
            def _(r):
                rv = jnp.full((_L,), 0, jnp.int32) + r
                for h in range(_D // _L):
                    v = plsc.load_gather(panel[b], [iota + (h * _L), rv])
                    stage[b][pl.ds(r * _D + h * _L, _L)] = v * _SCALE

        @pl.when(n_c > 0)
        def _():
            for b in range(_A_NBUF):
                @pl.when(b < n_c)
                def _():
                    load_desc(c_lo + b, b).start()

            def step(i, carry):
                b = lax.rem(i, _A_NBUF)

                def go(b, i=i):
                    c = c_lo + i
                    load_desc(c, b).wait()
                    transpose_panel(b)
                    sd = store_desc(c, b)
                    sd.start()
                    sd.wait()

                    @pl.when(i + _A_NBUF < n_c)
                    def _():
                        load_desc(c + _A_NBUF, b).start()

                for bb in range(_A_NBUF):
                    @pl.when(b == bb)
                    def _(bb=bb):
                        go(bb)
                return carry

            lax.fori_loop(0, n_c, step, 0)

        @pl.when(wid == _NW - 1)
        def _():
            pltpu.sync_copy(tin.at[:, pl.ds(_NFULL * _PANEL, _TAIL)], tpanel)
            for d in range(_D):
                for r0 in range(_TAIL // _L):
                    v = tpanel[d, pl.ds(r0 * _L, _L)]
                    plsc.store_scatter(
                        tstage, [iota_d + (r0 * _L * _D + d)], v * _SCALE)
            pltpu.sync_copy(
                tstage, tout.at[pl.ds(_NFULL * _PANEL * _D, _TAIL * _D)])

    return conv(table_t)


def _gather_rows(table_rm, idx_flat, n):
    per_w = n // _NW
    nchunk = per_w // _CHUNK
    nsuper = nchunk // _B_NBUF
    assert per_w * _NW == n and nchunk * _CHUNK == per_w
    assert nsuper * _B_NBUF == nchunk and nsuper >= 2

    mesh = plsc.VectorSubcoreMesh(
        core_axis_name="c", subcore_axis_name="s",
        num_cores=_NC, num_subcores=_NS)

    @functools.partial(
        pl.kernel,
        out_type=jax.ShapeDtypeStruct((n, _D), jnp.float32),
        mesh=mesh,
        scratch_types=[
            pltpu.VMEM((per_w,), jnp.int32),
            pltpu.VMEM((_B_NBUF, _CHUNK, _D), jnp.float32),
            pltpu.SemaphoreType.DMA,
            pltpu.SemaphoreType.DMA,
            pltpu.SemaphoreType.DMA,
            pltpu.SemaphoreType.DMA,
        ],
        compiler_params=pltpu.CompilerParams(use_tc_tiling_on_sc=False),
    )
    def emb(idx_hbm, table_hbm, out_hbm, idx_all, rows, sg0, sg1, so0, so1):
        sg = (sg0, sg1)
        so = (so0, so1)
        wid = lax.axis_index("s") * _NC + lax.axis_index("c")
        w_base = wid * per_w
        pltpu.sync_copy(idx_hbm.at[pl.ds(w_base, per_w)], idx_all)

        def gather_desc(c, b):
            return pltpu.make_async_copy(
                table_hbm.at[idx_all.at[pl.ds(c * _CHUNK, _CHUNK)]],
                rows.at[b], sg[b])

        def store_desc(c, b):
            return pltpu.make_async_copy(
                rows.at[b], out_hbm.at[pl.ds(w_base + c * _CHUNK, _CHUNK)],
                so[b])

        for b in range(_B_NBUF):
            gather_desc(b, b).start()

        @pl.loop(0, nsuper - 1)
        def super_step(g):
            for b in range(_B_NBUF):
                c = g * _B_NBUF + b
                gather_desc(c, b).wait()
                sd = store_desc(c, b)
                sd.start()
                sd.wait()
                gather_desc(c + _B_NBUF, b).start()

        for b in range(_B_NBUF):
            c_last = (nsuper - 1) * _B_NBUF + b
            gather_desc(c_last, b).wait()
            store_desc(c_last, b).start()
        for b in range(_B_NBUF):
            store_desc(nchunk - _B_NBUF + b, b).wait()

    return emb(idx_flat, table_rm)


def kernel(encoded_data, embedding_table):
    batch, seqlen = encoded_data.shape
    n = batch * seqlen

    table_t = jnp.transpose(embedding_table)
    table_flat = _transpose_scale_table(table_t)
    table_rm = jnp.reshape(table_flat, (_V, _D))

    idx_flat = encoded_data.reshape(n).astype(jnp.int32)
    out = _gather_rows(table_rm, idx_flat, n)
    return out.reshape(batch, seqlen, _D)
